# baseline (speedup 1.0000x reference)
import functools

import jax
import jax.numpy as jnp
from jax import lax
from jax.experimental import pallas as pl
from jax.experimental.pallas import tpu as pltpu
from jax.experimental.pallas import tpu_sc as plsc

_NC = 2
_NS = 16
_NW = _NC * _NS

_V = 1000000
_NBLK = _V // 128
_TAIL = _V - _NBLK * 128
_BPW = _NBLK // _NW
_NTAIL = _NBLK - _NW * _BPW

_CH = 128
_NFIELD = 26
_BATCH = 16384
_BW = _BATCH // _NW
_NCHF = _BW // _CH
_L = 16

_MESH = plsc.VectorSubcoreMesh(core_axis_name="c", subcore_axis_name="s")


@functools.partial(
    pl.kernel,
    out_type=jax.ShapeDtypeStruct((_V * 32,), jnp.float32),
    mesh=_MESH,
    scratch_types=[
        pltpu.VMEM((2, 32, 128), jnp.float32),
        pltpu.VMEM((2, 4096), jnp.float32),
        pltpu.SemaphoreType.DMA,
        pltpu.SemaphoreType.DMA,
        pltpu.SemaphoreType.DMA,
        pltpu.SemaphoreType.DMA,
    ],
    compiler_params=pltpu.CompilerParams(use_tc_tiling_on_sc=True,
                                         needs_layout_passes=False),
)
def _kt(tbl_t, tail_in, out, src_v, dst_v, si0, si1, so0, so1):
  wid = lax.axis_index("s") * _NC + lax.axis_index("c")
  lane = lax.iota(jnp.int32, _L)

  def fire_in(j, p, sem):
    pltpu.async_copy(tbl_t.at[:, pl.ds(j * 128, 128)], src_v.at[p], sem)

  def transpose_block(p, width):
    def tbody(m0, carry):
      for mm in range(8):
        m = m0 * 8 + mm
        d_vec = lane + _L * (mm % 2)
        c = m // 2
        vals = plsc.load_gather(src_v.at[p], [d_vec, jnp.full((_L,), 0,
                                                             jnp.int32) + c])
        dst_v[p, pl.ds(m0 * 128 + mm * 16, _L)] = vals
      return carry

    lax.fori_loop(0, width // 4, tbody, 0)

  def fire_out(j, p, sem, width):
    pltpu.async_copy(dst_v.at[p, pl.ds(0, width * 32)],
                     out.at[pl.ds(j * 4096, width * 32)], sem)

  def wait(sem, nbytes):
    pltpu.make_async_copy(out.at[pl.ds(0, nbytes // 4)],
                          dst_v.at[0, pl.ds(0, nbytes // 4)], sem).wait()

  fire_in(wid, 0, si0)
  fire_in(wid + _NW, 1, si1)

  def body(i, carry):
    t0 = 2 * i
    wait(si0, 16384)
    transpose_block(0, 128)
    fire_out(wid + _NW * t0, 0, so0, 128)
    wait(si1, 16384)
    transpose_block(1, 128)
    fire_out(wid + _NW * (t0 + 1), 1, so1, 128)
    wait(so0, 16384)
    fire_in(wid + _NW * (t0 + 2), 0, si0)
    wait(so1, 16384)
    fire_in(wid + _NW * (t0 + 3), 1, si1)
    return carry

  lax.fori_loop(0, _BPW // 2 - 1, body, 0)

  wait(si0, 16384)
  transpose_block(0, 128)
  fire_out(wid + _NW * (_BPW - 2), 0, so0, 128)
  wait(si1, 16384)
  transpose_block(1, 128)
  fire_out(wid + _NW * (_BPW - 1), 1, so1, 128)
  wait(so0, 16384)
  wait(so1, 16384)

  @pl.when(wid < _NTAIL)
  def _():
    j = _NW * _BPW + wid
    fire_in(j, 0, si0)
    wait(si0, 16384)
    transpose_block(0, 128)
    fire_out(j, 0, so0, 128)
    wait(so0, 16384)

  @pl.when(wid == _NTAIL)
  def _():
    pltpu.sync_copy(tail_in, dst_v.at[0, pl.ds(0, _TAIL * 32)])
    pltpu.sync_copy(dst_v.at[0, pl.ds(0, _TAIL * 32)],
                    out.at[pl.ds(_NBLK * 4096, _TAIL * 32)])


@functools.partial(jax.jit, static_argnums=(2,))
def _sc_gather(tbl_rm, idx, d):
  @functools.partial(
      pl.kernel,
      out_type=jax.ShapeDtypeStruct((_NFIELD, _BATCH, d), jnp.float32),
      mesh=_MESH,
      scratch_types=[
          pltpu.VMEM((_BW, _NFIELD), jnp.int32),
          pltpu.VMEM((_NFIELD, _BW), jnp.int32),
          pltpu.VMEM((2, _BW, d), jnp.float32),
          pltpu.SemaphoreType.DMA,
          pltpu.SemaphoreType.DMA,
          pltpu.SemaphoreType.DMA,
          pltpu.SemaphoreType.DMA,
      ],
      compiler_params=pltpu.CompilerParams(use_tc_tiling_on_sc=False,
                                           needs_layout_passes=False),
  )
  def k(table_hbm, idx_hbm, out_hbm, idx_raw, idx_v, rows_v,
        sem0, sem1, wsem0, wsem1):
    wid = lax.axis_index("s") * _NC + lax.axis_index("c")
    base = wid * _BW
    pltpu.sync_copy(idx_hbm.at[pl.ds(base, _BW)], idx_raw)

    lane = lax.iota(jnp.int32, _L)
    for j0 in range(_BW // _L):
      rows = j0 * _L + lane

      def tbody(f, carry):
        cols = jnp.full((_L,), 0, jnp.int32) + f
        vals = plsc.load_gather(idx_raw, [rows, cols])
        idx_v[f, pl.ds(j0 * _L, _L)] = vals
        return carry

      lax.fori_loop(0, _NFIELD, tbody, 0)

    def fire_field(f, p, sem):
      for c in range(_NCHF):
        pltpu.async_copy(
            table_hbm.at[idx_v.at[f, pl.ds(c * _CH, _CH)]],
            rows_v.at[p, pl.ds(c * _CH, _CH)],
            sem,
        )

    def drain_field(p, sem):
      pltpu.make_async_copy(
          out_hbm.at[0, pl.ds(base, _BW)], rows_v.at[p], sem).wait()

    fire_field(0, 0, sem0)
    fire_field(1, 1, sem1)

    def body(i, carry):
      f = 2 * i
      drain_field(0, sem0)
      w0 = pltpu.async_copy(rows_v.at[0], out_hbm.at[f, pl.ds(base, _BW)],
                            wsem0)
      drain_field(1, sem1)
      w1 = pltpu.async_copy(rows_v.at[1], out_hbm.at[f + 1, pl.ds(base, _BW)],
                            wsem1)
      w0.wait()
      fire_field(f + 2, 0, sem0)
      w1.wait()
      fire_field(f + 3, 1, sem1)
      return carry

    lax.fori_loop(0, _NFIELD // 2 - 1, body, 0)

    drain_field(0, sem0)
    pltpu.async_copy(rows_v.at[0], out_hbm.at[_NFIELD - 2, pl.ds(base, _BW)],
                     wsem0)
    drain_field(1, sem1)
    pltpu.async_copy(rows_v.at[1], out_hbm.at[_NFIELD - 1, pl.ds(base, _BW)],
                     wsem1)
    pltpu.make_async_copy(rows_v.at[0], out_hbm.at[0, pl.ds(base, _BW)],
                          wsem0).wait()
    pltpu.make_async_copy(rows_v.at[1], out_hbm.at[0, pl.ds(base, _BW)],
                          wsem1).wait()

  return k(tbl_rm, idx)


def kernel(inputs, embedding):
  b, f = inputs.shape
  d = embedding.shape[1]
  idx = inputs if inputs.dtype == jnp.int32 else inputs.astype(jnp.int32)
  tail = embedding[_NBLK * 128:].reshape(-1)
  tbl_rm = _kt(embedding.T, tail).reshape(_V, d)
  out = _sc_gather(tbl_rm, idx, d)
  return out.transpose(1, 0, 2)


# speedup vs baseline: 1.4228x; 1.4228x over previous
import functools

import jax
import jax.numpy as jnp
from jax import lax
from jax.experimental import pallas as pl
from jax.experimental.pallas import tpu as pltpu
from jax.experimental.pallas import tpu_sc as plsc

_NC = 2
_NS = 16
_NW = _NC * _NS

_V = 1000000
_NBLK = _V // 128
_TAIL = _V - _NBLK * 128
_BPW = _NBLK // _NW
_NTAIL = _NBLK - _NW * _BPW

_CH = 128
_NFIELD = 26
_BATCH = 16384
_BW = _BATCH // _NW
_NCHF = _BW // _CH
_L = 16

_MESH = plsc.VectorSubcoreMesh(core_axis_name="c", subcore_axis_name="s")


@functools.partial(
    pl.kernel,
    out_type=jax.ShapeDtypeStruct((_V * 32,), jnp.float32),
    mesh=_MESH,
    scratch_types=[
        pltpu.VMEM((2, 32, 128), jnp.float32),
        pltpu.VMEM((2, 4096), jnp.float32),
        pltpu.SemaphoreType.DMA,
        pltpu.SemaphoreType.DMA,
        pltpu.SemaphoreType.DMA,
        pltpu.SemaphoreType.DMA,
    ],
    compiler_params=pltpu.CompilerParams(use_tc_tiling_on_sc=True,
                                         needs_layout_passes=False),
)
def _kt(tbl_t, tail_in, out, src_v, dst_v, si0, si1, so0, so1):
  wid = lax.axis_index("s") * _NC + lax.axis_index("c")
  lane = lax.iota(jnp.int32, _L)

  def fire_in(j, p, sem):
    pltpu.async_copy(tbl_t.at[:, pl.ds(j * 128, 128)], src_v.at[p], sem)

  def transpose_block(p, width):
    pvec = jnp.full((_L,), 0, jnp.int32) + p

    def tbody(q, carry):
      for k in range(_L):
        perm = (lane + k) & (_L - 1)
        cvec = q * _L + lane
        dflat = q * 512 + lane * 32 + perm
        for h in range(2):
          vals = plsc.load_gather(src_v, [pvec, h * _L + perm, cvec])
          plsc.store_scatter(dst_v, [pvec, dflat + h * _L], vals)
      return carry

    lax.fori_loop(0, width // _L, tbody, 0)

  def fire_out(j, p, sem, width):
    pltpu.async_copy(dst_v.at[p, pl.ds(0, width * 32)],
                     out.at[pl.ds(j * 4096, width * 32)], sem)

  def wait(sem, nbytes):
    pltpu.make_async_copy(out.at[pl.ds(0, nbytes // 4)],
                          dst_v.at[0, pl.ds(0, nbytes // 4)], sem).wait()

  fire_in(wid, 0, si0)
  fire_in(wid + _NW, 1, si1)

  def body(i, carry):
    t0 = 2 * i
    wait(si0, 16384)
    transpose_block(0, 128)
    fire_out(wid + _NW * t0, 0, so0, 128)
    wait(si1, 16384)
    transpose_block(1, 128)
    fire_out(wid + _NW * (t0 + 1), 1, so1, 128)
    wait(so0, 16384)
    fire_in(wid + _NW * (t0 + 2), 0, si0)
    wait(so1, 16384)
    fire_in(wid + _NW * (t0 + 3), 1, si1)
    return carry

  lax.fori_loop(0, _BPW // 2 - 1, body, 0)

  wait(si0, 16384)
  transpose_block(0, 128)
  fire_out(wid + _NW * (_BPW - 2), 0, so0, 128)
  wait(si1, 16384)
  transpose_block(1, 128)
  fire_out(wid + _NW * (_BPW - 1), 1, so1, 128)
  wait(so0, 16384)
  wait(so1, 16384)

  @pl.when(wid < _NTAIL)
  def _():
    j = _NW * _BPW + wid
    fire_in(j, 0, si0)
    wait(si0, 16384)
    transpose_block(0, 128)
    fire_out(j, 0, so0, 128)
    wait(so0, 16384)

  @pl.when(wid == _NTAIL)
  def _():
    pltpu.sync_copy(tail_in, dst_v.at[0, pl.ds(0, _TAIL * 32)])
    pltpu.sync_copy(dst_v.at[0, pl.ds(0, _TAIL * 32)],
                    out.at[pl.ds(_NBLK * 4096, _TAIL * 32)])


@functools.partial(jax.jit, static_argnums=(2,))
def _sc_gather(tbl_rm, idx, d):
  @functools.partial(
      pl.kernel,
      out_type=jax.ShapeDtypeStruct((_NFIELD, _BATCH, d), jnp.float32),
      mesh=_MESH,
      scratch_types=[
          pltpu.VMEM((_BW, _NFIELD), jnp.int32),
          pltpu.VMEM((_NFIELD, _BW), jnp.int32),
          pltpu.VMEM((2, _BW, d), jnp.float32),
          pltpu.SemaphoreType.DMA,
          pltpu.SemaphoreType.DMA,
          pltpu.SemaphoreType.DMA,
          pltpu.SemaphoreType.DMA,
      ],
      compiler_params=pltpu.CompilerParams(use_tc_tiling_on_sc=False,
                                           needs_layout_passes=False),
  )
  def k(table_hbm, idx_hbm, out_hbm, idx_raw, idx_v, rows_v,
        sem0, sem1, wsem0, wsem1):
    wid = lax.axis_index("s") * _NC + lax.axis_index("c")
    base = wid * _BW
    pltpu.sync_copy(idx_hbm.at[pl.ds(base, _BW)], idx_raw)

    lane = lax.iota(jnp.int32, _L)
    for j0 in range(_BW // _L):
      rows = j0 * _L + lane

      def tbody(f, carry):
        cols = jnp.full((_L,), 0, jnp.int32) + f
        vals = plsc.load_gather(idx_raw, [rows, cols])
        idx_v[f, pl.ds(j0 * _L, _L)] = vals
        return carry

      lax.fori_loop(0, _NFIELD, tbody, 0)

    def fire_field(f, p, sem):
      for c in range(_NCHF):
        pltpu.async_copy(
            table_hbm.at[idx_v.at[f, pl.ds(c * _CH, _CH)]],
            rows_v.at[p, pl.ds(c * _CH, _CH)],
            sem,
        )

    def drain_field(p, sem):
      pltpu.make_async_copy(
          out_hbm.at[0, pl.ds(base, _BW)], rows_v.at[p], sem).wait()

    fire_field(0, 0, sem0)
    fire_field(1, 1, sem1)

    def body(i, carry):
      f = 2 * i
      drain_field(0, sem0)
      w0 = pltpu.async_copy(rows_v.at[0], out_hbm.at[f, pl.ds(base, _BW)],
                            wsem0)
      drain_field(1, sem1)
      w1 = pltpu.async_copy(rows_v.at[1], out_hbm.at[f + 1, pl.ds(base, _BW)],
                            wsem1)
      w0.wait()
      fire_field(f + 2, 0, sem0)
      w1.wait()
      fire_field(f + 3, 1, sem1)
      return carry

    lax.fori_loop(0, _NFIELD // 2 - 1, body, 0)

    drain_field(0, sem0)
    pltpu.async_copy(rows_v.at[0], out_hbm.at[_NFIELD - 2, pl.ds(base, _BW)],
                     wsem0)
    drain_field(1, sem1)
    pltpu.async_copy(rows_v.at[1], out_hbm.at[_NFIELD - 1, pl.ds(base, _BW)],
                     wsem1)
    pltpu.make_async_copy(rows_v.at[0], out_hbm.at[0, pl.ds(base, _BW)],
                          wsem0).wait()
    pltpu.make_async_copy(rows_v.at[1], out_hbm.at[0, pl.ds(base, _BW)],
                          wsem1).wait()

  return k(tbl_rm, idx)


def kernel(inputs, embedding):
  b, f = inputs.shape
  d = embedding.shape[1]
  idx = inputs if inputs.dtype == jnp.int32 else inputs.astype(jnp.int32)
  tail = embedding[_NBLK * 128:].reshape(-1)
  tbl_rm = _kt(embedding.T, tail).reshape(_V, d)
  out = _sc_gather(tbl_rm, idx, d)
  return out.transpose(1, 0, 2)


# speedup vs baseline: 1.4300x; 1.0050x over previous
import functools

import jax
import jax.numpy as jnp
from jax import lax
from jax.experimental import pallas as pl
from jax.experimental.pallas import tpu as pltpu
from jax.experimental.pallas import tpu_sc as plsc

_NC = 2
_NS = 16
_NW = _NC * _NS

_V = 1000000
_NBLK = _V // 128
_TAIL = _V - _NBLK * 128
_BPW = _NBLK // _NW
_NTAIL = _NBLK - _NW * _BPW

_CH = 128
_NFIELD = 26
_BATCH = 16384
_BW = _BATCH // _NW
_NCHF = _BW // _CH
_L = 16

_MESH = plsc.VectorSubcoreMesh(core_axis_name="c", subcore_axis_name="s")


@functools.partial(
    pl.kernel,
    out_type=jax.ShapeDtypeStruct((_V * 32,), jnp.float32),
    mesh=_MESH,
    scratch_types=[
        pltpu.VMEM((2, 32, 128), jnp.float32),
        pltpu.VMEM((2, 4096), jnp.float32),
        pltpu.SemaphoreType.DMA,
        pltpu.SemaphoreType.DMA,
        pltpu.SemaphoreType.DMA,
        pltpu.SemaphoreType.DMA,
    ],
    compiler_params=pltpu.CompilerParams(use_tc_tiling_on_sc=True,
                                         needs_layout_passes=False),
)
def _kt(tbl_t, tail_in, out, src_v, dst_v, si0, si1, so0, so1):
  wid = lax.axis_index("s") * _NC + lax.axis_index("c")
  lane = lax.iota(jnp.int32, _L)

  def fire_in(j, p, sem):
    pltpu.async_copy(tbl_t.at[:, pl.ds(j * 128, 128)], src_v.at[p], sem)

  def transpose_block(p, width):
    pvec = jnp.full((_L,), 0, jnp.int32) + p

    @plsc.parallel_loop(0, width // _L, unroll=2)
    def _(q):
      for k in range(_L):
        perm = (lane + k) & (_L - 1)
        cvec = q * _L + lane
        dflat = q * 512 + lane * 32 + perm
        for h in range(2):
          vals = plsc.load_gather(src_v, [pvec, h * _L + perm, cvec])
          plsc.store_scatter(dst_v, [pvec, dflat + h * _L], vals)

  def fire_out(j, p, sem, width):
    pltpu.async_copy(dst_v.at[p, pl.ds(0, width * 32)],
                     out.at[pl.ds(j * 4096, width * 32)], sem)

  def wait(sem, nbytes):
    pltpu.make_async_copy(out.at[pl.ds(0, nbytes // 4)],
                          dst_v.at[0, pl.ds(0, nbytes // 4)], sem).wait()

  fire_in(wid, 0, si0)
  fire_in(wid + _NW, 1, si1)

  def body(i, carry):
    t0 = 2 * i
    wait(si0, 16384)
    transpose_block(0, 128)
    fire_out(wid + _NW * t0, 0, so0, 128)
    wait(si1, 16384)
    transpose_block(1, 128)
    fire_out(wid + _NW * (t0 + 1), 1, so1, 128)
    wait(so0, 16384)
    fire_in(wid + _NW * (t0 + 2), 0, si0)
    wait(so1, 16384)
    fire_in(wid + _NW * (t0 + 3), 1, si1)
    return carry

  lax.fori_loop(0, _BPW // 2 - 1, body, 0)

  wait(si0, 16384)
  transpose_block(0, 128)
  fire_out(wid + _NW * (_BPW - 2), 0, so0, 128)
  wait(si1, 16384)
  transpose_block(1, 128)
  fire_out(wid + _NW * (_BPW - 1), 1, so1, 128)
  wait(so0, 16384)
  wait(so1, 16384)

  @pl.when(wid < _NTAIL)
  def _():
    j = _NW * _BPW + wid
    fire_in(j, 0, si0)
    wait(si0, 16384)
    transpose_block(0, 128)
    fire_out(j, 0, so0, 128)
    wait(so0, 16384)

  @pl.when(wid == _NTAIL)
  def _():
    pltpu.sync_copy(tail_in, dst_v.at[0, pl.ds(0, _TAIL * 32)])
    pltpu.sync_copy(dst_v.at[0, pl.ds(0, _TAIL * 32)],
                    out.at[pl.ds(_NBLK * 4096, _TAIL * 32)])


@functools.partial(jax.jit, static_argnums=(2,))
def _sc_gather(tbl_rm, idx, d):
  @functools.partial(
      pl.kernel,
      out_type=jax.ShapeDtypeStruct((_NFIELD, _BATCH, d), jnp.float32),
      mesh=_MESH,
      scratch_types=[
          pltpu.VMEM((_BW, _NFIELD), jnp.int32),
          pltpu.VMEM((_NFIELD, _BW), jnp.int32),
          pltpu.VMEM((2, _BW, d), jnp.float32),
          pltpu.SemaphoreType.DMA,
          pltpu.SemaphoreType.DMA,
          pltpu.SemaphoreType.DMA,
          pltpu.SemaphoreType.DMA,
      ],
      compiler_params=pltpu.CompilerParams(use_tc_tiling_on_sc=False,
                                           needs_layout_passes=False),
  )
  def k(table_hbm, idx_hbm, out_hbm, idx_raw, idx_v, rows_v,
        sem0, sem1, wsem0, wsem1):
    wid = lax.axis_index("s") * _NC + lax.axis_index("c")
    base = wid * _BW
    pltpu.sync_copy(idx_hbm.at[pl.ds(base, _BW)], idx_raw)

    lane = lax.iota(jnp.int32, _L)
    for j0 in range(_BW // _L):
      rows = j0 * _L + lane

      def tbody(f, carry):
        cols = jnp.full((_L,), 0, jnp.int32) + f
        vals = plsc.load_gather(idx_raw, [rows, cols])
        idx_v[f, pl.ds(j0 * _L, _L)] = vals
        return carry

      lax.fori_loop(0, _NFIELD, tbody, 0)

    def fire_field(f, p, sem):
      for c in range(_NCHF):
        pltpu.async_copy(
            table_hbm.at[idx_v.at[f, pl.ds(c * _CH, _CH)]],
            rows_v.at[p, pl.ds(c * _CH, _CH)],
            sem,
        )

    def drain_field(p, sem):
      pltpu.make_async_copy(
          out_hbm.at[0, pl.ds(base, _BW)], rows_v.at[p], sem).wait()

    fire_field(0, 0, sem0)
    fire_field(1, 1, sem1)

    def body(i, carry):
      f = 2 * i
      drain_field(0, sem0)
      w0 = pltpu.async_copy(rows_v.at[0], out_hbm.at[f, pl.ds(base, _BW)],
                            wsem0)
      drain_field(1, sem1)
      w1 = pltpu.async_copy(rows_v.at[1], out_hbm.at[f + 1, pl.ds(base, _BW)],
                            wsem1)
      w0.wait()
      fire_field(f + 2, 0, sem0)
      w1.wait()
      fire_field(f + 3, 1, sem1)
      return carry

    lax.fori_loop(0, _NFIELD // 2 - 1, body, 0)

    drain_field(0, sem0)
    pltpu.async_copy(rows_v.at[0], out_hbm.at[_NFIELD - 2, pl.ds(base, _BW)],
                     wsem0)
    drain_field(1, sem1)
    pltpu.async_copy(rows_v.at[1], out_hbm.at[_NFIELD - 1, pl.ds(base, _BW)],
                     wsem1)
    pltpu.make_async_copy(rows_v.at[0], out_hbm.at[0, pl.ds(base, _BW)],
                          wsem0).wait()
    pltpu.make_async_copy(rows_v.at[1], out_hbm.at[0, pl.ds(base, _BW)],
                          wsem1).wait()

  return k(tbl_rm, idx)


def kernel(inputs, embedding):
  b, f = inputs.shape
  d = embedding.shape[1]
  idx = inputs if inputs.dtype == jnp.int32 else inputs.astype(jnp.int32)
  tail = embedding[_NBLK * 128:].reshape(-1)
  tbl_rm = _kt(embedding.T, tail).reshape(_V, d)
  out = _sc_gather(tbl_rm, idx, d)
  return out.transpose(1, 0, 2)


# speedup vs baseline: 2.2493x; 1.5730x over previous
import functools

import jax
import jax.numpy as jnp
from jax import lax
from jax.experimental import pallas as pl
from jax.experimental.pallas import tpu as pltpu
from jax.experimental.pallas import tpu_sc as plsc

_NC = 2
_NS = 16
_NW = _NC * _NS

_V = 1000000
_NBLK = _V // 128
_TAIL = _V - _NBLK * 128
_BPW = _NBLK // _NW
_NTAIL = _NBLK - _NW * _BPW

_CH = 128
_NFIELD = 26
_BATCH = 16384
_BW = _BATCH // _NW
_NCHF = _BW // _CH
_L = 16

_MESH = plsc.VectorSubcoreMesh(core_axis_name="c", subcore_axis_name="s")


@functools.partial(
    pl.kernel,
    out_type=jax.ShapeDtypeStruct((_V * 32,), jnp.float32),
    mesh=_MESH,
    scratch_types=[
        pltpu.VMEM((32, 128), jnp.float32),
        pltpu.VMEM((32, 128), jnp.float32),
        pltpu.VMEM((4096,), jnp.float32),
        pltpu.VMEM((4096,), jnp.float32),
        pltpu.SemaphoreType.DMA,
        pltpu.SemaphoreType.DMA,
        pltpu.SemaphoreType.DMA,
        pltpu.SemaphoreType.DMA,
    ],
    compiler_params=pltpu.CompilerParams(use_tc_tiling_on_sc=True,
                                         needs_layout_passes=False),
)
def _kt(tbl_t, tail_in, out, src_v0, src_v1, dst_v0, dst_v1,
        si0, si1, so0, so1):
  wid = lax.axis_index("s") * _NC + lax.axis_index("c")
  lane = lax.iota(jnp.int32, _L)
  _SRC = (src_v0, src_v1)
  _DST = (dst_v0, dst_v1)

  def fire_in(j, p, sem):
    pltpu.async_copy(tbl_t.at[:, pl.ds(j * 128, 128)], _SRC[p], sem)

  def transpose_block(p, width):
    src, dst = _SRC[p], _DST[p]

    @plsc.parallel_loop(0, width // _L, unroll=2)
    def _(q):
      for k in range(_L):
        perm = (lane + k) & (_L - 1)
        cvec = q * _L + lane
        dflat = q * 512 + lane * 32 + perm
        for h in range(2):
          vals = plsc.load_gather(src, [h * _L + perm, cvec])
          plsc.store_scatter(dst, [dflat + h * _L], vals)

  def fire_out(j, p, sem, width):
    pltpu.async_copy(_DST[p].at[pl.ds(0, width * 32)],
                     out.at[pl.ds(j * 4096, width * 32)], sem)

  def wait(sem, nbytes):
    pltpu.make_async_copy(out.at[pl.ds(0, nbytes // 4)],
                          dst_v0.at[pl.ds(0, nbytes // 4)], sem).wait()

  fire_in(wid, 0, si0)
  fire_in(wid + _NW, 1, si1)

  def body(i, carry):
    t0 = 2 * i
    wait(si0, 16384)
    transpose_block(0, 128)
    fire_out(wid + _NW * t0, 0, so0, 128)
    wait(si1, 16384)
    transpose_block(1, 128)
    fire_out(wid + _NW * (t0 + 1), 1, so1, 128)
    wait(so0, 16384)
    fire_in(wid + _NW * (t0 + 2), 0, si0)
    wait(so1, 16384)
    fire_in(wid + _NW * (t0 + 3), 1, si1)
    return carry

  lax.fori_loop(0, _BPW // 2 - 1, body, 0)

  wait(si0, 16384)
  transpose_block(0, 128)
  fire_out(wid + _NW * (_BPW - 2), 0, so0, 128)
  wait(si1, 16384)
  transpose_block(1, 128)
  fire_out(wid + _NW * (_BPW - 1), 1, so1, 128)
  wait(so0, 16384)
  wait(so1, 16384)

  @pl.when(wid < _NTAIL)
  def _():
    j = _NW * _BPW + wid
    fire_in(j, 0, si0)
    wait(si0, 16384)
    transpose_block(0, 128)
    fire_out(j, 0, so0, 128)
    wait(so0, 16384)

  @pl.when(wid == _NTAIL)
  def _():
    pltpu.sync_copy(tail_in, dst_v0.at[pl.ds(0, _TAIL * 32)])
    pltpu.sync_copy(dst_v0.at[pl.ds(0, _TAIL * 32)],
                    out.at[pl.ds(_NBLK * 4096, _TAIL * 32)])


@functools.partial(jax.jit, static_argnums=(2,))
def _sc_gather(tbl_rm, idx, d):
  @functools.partial(
      pl.kernel,
      out_type=jax.ShapeDtypeStruct((_NFIELD, _BATCH, d), jnp.float32),
      mesh=_MESH,
      scratch_types=[
          pltpu.VMEM((_BW, _NFIELD), jnp.int32),
          pltpu.VMEM((_NFIELD, _BW), jnp.int32),
          pltpu.VMEM((2, _BW, d), jnp.float32),
          pltpu.SemaphoreType.DMA,
          pltpu.SemaphoreType.DMA,
          pltpu.SemaphoreType.DMA,
          pltpu.SemaphoreType.DMA,
      ],
      compiler_params=pltpu.CompilerParams(use_tc_tiling_on_sc=False,
                                           needs_layout_passes=False),
  )
  def k(table_hbm, idx_hbm, out_hbm, idx_raw, idx_v, rows_v,
        sem0, sem1, wsem0, wsem1):
    wid = lax.axis_index("s") * _NC + lax.axis_index("c")
    base = wid * _BW
    pltpu.sync_copy(idx_hbm.at[pl.ds(base, _BW)], idx_raw)

    lane = lax.iota(jnp.int32, _L)
    for j0 in range(_BW // _L):
      rows = j0 * _L + lane

      def tbody(f, carry):
        cols = jnp.full((_L,), 0, jnp.int32) + f
        vals = plsc.load_gather(idx_raw, [rows, cols])
        idx_v[f, pl.ds(j0 * _L, _L)] = vals
        return carry

      lax.fori_loop(0, _NFIELD, tbody, 0)

    def fire_field(f, p, sem):
      for c in range(_NCHF):
        pltpu.async_copy(
            table_hbm.at[idx_v.at[f, pl.ds(c * _CH, _CH)]],
            rows_v.at[p, pl.ds(c * _CH, _CH)],
            sem,
        )

    def drain_field(p, sem):
      pltpu.make_async_copy(
          out_hbm.at[0, pl.ds(base, _BW)], rows_v.at[p], sem).wait()

    fire_field(0, 0, sem0)
    fire_field(1, 1, sem1)

    def body(i, carry):
      f = 2 * i
      drain_field(0, sem0)
      w0 = pltpu.async_copy(rows_v.at[0], out_hbm.at[f, pl.ds(base, _BW)],
                            wsem0)
      drain_field(1, sem1)
      w1 = pltpu.async_copy(rows_v.at[1], out_hbm.at[f + 1, pl.ds(base, _BW)],
                            wsem1)
      w0.wait()
      fire_field(f + 2, 0, sem0)
      w1.wait()
      fire_field(f + 3, 1, sem1)
      return carry

    lax.fori_loop(0, _NFIELD // 2 - 1, body, 0)

    drain_field(0, sem0)
    pltpu.async_copy(rows_v.at[0], out_hbm.at[_NFIELD - 2, pl.ds(base, _BW)],
                     wsem0)
    drain_field(1, sem1)
    pltpu.async_copy(rows_v.at[1], out_hbm.at[_NFIELD - 1, pl.ds(base, _BW)],
                     wsem1)
    pltpu.make_async_copy(rows_v.at[0], out_hbm.at[0, pl.ds(base, _BW)],
                          wsem0).wait()
    pltpu.make_async_copy(rows_v.at[1], out_hbm.at[0, pl.ds(base, _BW)],
                          wsem1).wait()

  return k(tbl_rm, idx)


def kernel(inputs, embedding):
  b, f = inputs.shape
  d = embedding.shape[1]
  idx = inputs if inputs.dtype == jnp.int32 else inputs.astype(jnp.int32)
  tail = embedding[_NBLK * 128:].reshape(-1)
  tbl_rm = _kt(embedding.T, tail).reshape(_V, d)
  out = _sc_gather(tbl_rm, idx, d)
  return out.transpose(1, 0, 2)


# speedup vs baseline: 3.4990x; 1.5556x over previous
import functools

import jax
import jax.numpy as jnp
from jax import lax
from jax.experimental import pallas as pl
from jax.experimental.pallas import tpu as pltpu
from jax.experimental.pallas import tpu_sc as plsc

_NC = 2
_NS = 16
_NW = _NC * _NS

_V = 1000000
_NBLK = _V // 128
_TAIL = _V - _NBLK * 128
_BPW = _NBLK // _NW
_NTAIL = _NBLK - _NW * _BPW

_CH = 128
_NFIELD = 26
_BATCH = 16384
_BW = _BATCH // _NW
_NCHF = _BW // _CH
_L = 16

_MESH = plsc.VectorSubcoreMesh(core_axis_name="c", subcore_axis_name="s")


@functools.partial(
    pl.kernel,
    out_type=jax.ShapeDtypeStruct((_V * 32,), jnp.float32),
    mesh=_MESH,
    scratch_types=[
        pltpu.VMEM((32, 128), jnp.float32),
        pltpu.VMEM((32, 128), jnp.float32),
        pltpu.VMEM((4096,), jnp.float32),
        pltpu.VMEM((4096,), jnp.float32),
        pltpu.SemaphoreType.DMA,
        pltpu.SemaphoreType.DMA,
        pltpu.SemaphoreType.DMA,
        pltpu.SemaphoreType.DMA,
    ],
    compiler_params=pltpu.CompilerParams(use_tc_tiling_on_sc=True,
                                         needs_layout_passes=False),
)
def _kt(tbl_t, tail_in, out, src_v0, src_v1, dst_v0, dst_v1,
        si0, si1, so0, so1):
  wid = lax.axis_index("s") * _NC + lax.axis_index("c")
  lane = lax.iota(jnp.int32, _L)
  _SRC = (src_v0, src_v1)
  _DST = (dst_v0, dst_v1)

  def fire_in(j, p, sem):
    pltpu.async_copy(tbl_t.at[:, pl.ds(j * 128, 128)], _SRC[p], sem)

  def transpose_block(p, width):
    src, dst = _SRC[p], _DST[p]

    @plsc.parallel_loop(0, width // _L, unroll=2)
    def _(q):
      for k in range(_L):
        perm = (lane + k) & (_L - 1)
        cvec = q * _L + lane
        dflat = q * 512 + lane * 32 + perm
        for h in range(2):
          vals = plsc.load_gather(src, [h * _L + perm, cvec])
          plsc.store_scatter(dst, [dflat + h * _L], vals)

  def fire_out(j, p, sem, width):
    pltpu.async_copy(_DST[p].at[pl.ds(0, width * 32)],
                     out.at[pl.ds(j * 4096, width * 32)], sem)

  def wait(sem, nbytes):
    pltpu.make_async_copy(out.at[pl.ds(0, nbytes // 4)],
                          dst_v0.at[pl.ds(0, nbytes // 4)], sem).wait()

  fire_in(wid, 0, si0)
  fire_in(wid + _NW, 1, si1)

  def body(i, carry):
    t0 = 2 * i
    wait(si0, 16384)
    transpose_block(0, 128)
    fire_out(wid + _NW * t0, 0, so0, 128)
    wait(si1, 16384)
    transpose_block(1, 128)
    fire_out(wid + _NW * (t0 + 1), 1, so1, 128)
    wait(so0, 16384)
    fire_in(wid + _NW * (t0 + 2), 0, si0)
    wait(so1, 16384)
    fire_in(wid + _NW * (t0 + 3), 1, si1)
    return carry

  lax.fori_loop(0, _BPW // 2 - 1, body, 0)

  wait(si0, 16384)
  transpose_block(0, 128)
  fire_out(wid + _NW * (_BPW - 2), 0, so0, 128)
  wait(si1, 16384)
  transpose_block(1, 128)
  fire_out(wid + _NW * (_BPW - 1), 1, so1, 128)
  wait(so0, 16384)
  wait(so1, 16384)

  @pl.when(wid < _NTAIL)
  def _():
    j = _NW * _BPW + wid
    fire_in(j, 0, si0)
    wait(si0, 16384)
    transpose_block(0, 128)
    fire_out(j, 0, so0, 128)
    wait(so0, 16384)

  @pl.when(wid == _NTAIL)
  def _():
    pltpu.sync_copy(tail_in, dst_v0.at[pl.ds(0, _TAIL * 32)])
    pltpu.sync_copy(dst_v0.at[pl.ds(0, _TAIL * 32)],
                    out.at[pl.ds(_NBLK * 4096, _TAIL * 32)])


@functools.partial(jax.jit, static_argnums=(2,))
def _sc_gather(tbl_rm, idx, d):
  @functools.partial(
      pl.kernel,
      out_type=jax.ShapeDtypeStruct((_NFIELD, d // 8, _BATCH // 128, 8, 128),
                                    jnp.float32),
      mesh=_MESH,
      scratch_types=[
          pltpu.VMEM((_BW, _NFIELD), jnp.int32),
          pltpu.VMEM((_NFIELD, _BW), jnp.int32),
          pltpu.VMEM((_BW, d), jnp.float32),
          pltpu.VMEM((_BW, d), jnp.float32),
          pltpu.VMEM((4, 4, 8, 128), jnp.float32),
          pltpu.VMEM((4, 4, 8, 128), jnp.float32),
          pltpu.SemaphoreType.DMA,
          pltpu.SemaphoreType.DMA,
          pltpu.SemaphoreType.DMA,
          pltpu.SemaphoreType.DMA,
      ],
      compiler_params=pltpu.CompilerParams(use_tc_tiling_on_sc=False,
                                           needs_layout_passes=False),
  )
  def k(table_hbm, idx_hbm, out_hbm, idx_raw, idx_v, rows_v0, rows_v1,
        stg0, stg1, sem0, sem1, wsem0, wsem1):
    _ROWS = (rows_v0, rows_v1)
    _STG = (stg0, stg1)
    wid = lax.axis_index("s") * _NC + lax.axis_index("c")
    base = wid * _BW
    pltpu.sync_copy(idx_hbm.at[pl.ds(base, _BW)], idx_raw)

    lane = lax.iota(jnp.int32, _L)
    for j0 in range(_BW // _L):
      rows = j0 * _L + lane

      def tbody(f, carry):
        cols = jnp.full((_L,), 0, jnp.int32) + f
        vals = plsc.load_gather(idx_raw, [rows, cols])
        idx_v[f, pl.ds(j0 * _L, _L)] = vals
        return carry

      lax.fori_loop(0, _NFIELD, tbody, 0)

    def fire_field(f, p, sem):
      for c in range(_NCHF):
        pltpu.async_copy(
            table_hbm.at[idx_v.at[f, pl.ds(c * _CH, _CH)]],
            _ROWS[p].at[pl.ds(c * _CH, _CH)],
            sem,
        )

    def drain_field(p, sem):
      pltpu.make_async_copy(
          out_hbm.at[0, :, pl.ds(0, 4)], _STG[p], sem).wait()

    def transpose_field(p):
      rows, stg = _ROWS[p], _STG[p]

      @plsc.parallel_loop(0, _BW // _L, unroll=2)
      def _(q):
        bvec = q * _L + lane
        tcl = q // 8
        cvec = (q % 8) * _L + lane
        for k in range(_L):
          perm = (lane + k) & (_L - 1)
          svec = perm % 8
          trb = perm // 8
          for h in range(2):
            vals = plsc.load_gather(rows, [bvec, h * _L + perm])
            plsc.store_scatter(stg, [trb + 2 * h, tcl + svec * 0, svec, cvec],
                               vals)

    def fire_out(f, p, sem):
      for tr in range(4):
        pltpu.async_copy(_STG[p].at[tr],
                         out_hbm.at[f, tr, pl.ds(4 * wid, 4)], sem)

    def wait_out(p, sem):
      pltpu.make_async_copy(out_hbm.at[0, :, pl.ds(0, 4)], _STG[p],
                            sem).wait()

    fire_field(0, 0, sem0)
    fire_field(1, 1, sem1)

    drain_field(0, sem0)
    transpose_field(0)
    fire_out(0, 0, wsem0)
    fire_field(2, 0, sem0)
    drain_field(1, sem1)
    transpose_field(1)
    fire_out(1, 1, wsem1)
    fire_field(3, 1, sem1)

    def body(i, carry):
      f = 2 * i
      drain_field(0, sem0)
      wait_out(0, wsem0)
      transpose_field(0)
      fire_out(f, 0, wsem0)
      fire_field(f + 2, 0, sem0)
      drain_field(1, sem1)
      wait_out(1, wsem1)
      transpose_field(1)
      fire_out(f + 1, 1, wsem1)
      fire_field(f + 3, 1, sem1)
      return carry

    lax.fori_loop(1, _NFIELD // 2 - 1, body, 0)

    drain_field(0, sem0)
    wait_out(0, wsem0)
    transpose_field(0)
    fire_out(_NFIELD - 2, 0, wsem0)
    drain_field(1, sem1)
    wait_out(1, wsem1)
    transpose_field(1)
    fire_out(_NFIELD - 1, 1, wsem1)
    wait_out(0, wsem0)
    wait_out(1, wsem1)

  return k(tbl_rm, idx)


def kernel(inputs, embedding):
  b, f = inputs.shape
  d = embedding.shape[1]
  idx = inputs if inputs.dtype == jnp.int32 else inputs.astype(jnp.int32)
  tail = embedding[_NBLK * 128:].reshape(-1)
  tbl_rm = _kt(embedding.T, tail).reshape(_V, d)
  out5 = _sc_gather(tbl_rm, idx, d)
  out = out5.transpose(0, 1, 3, 2, 4).reshape(f, d, b)
  return out.transpose(2, 0, 1)


# speedup vs baseline: 3.7869x; 1.0823x over previous
import functools

import jax
import jax.numpy as jnp
from jax import lax
from jax.experimental import pallas as pl
from jax.experimental.pallas import tpu as pltpu
from jax.experimental.pallas import tpu_sc as plsc

_NC = 2
_NS = 16
_NW = _NC * _NS

_V = 1000000
_NBLK = _V // 128
_TAIL = _V - _NBLK * 128
_BPW = _NBLK // _NW
_NTAIL = _NBLK - _NW * _BPW

_CH = 128
_NFIELD = 26
_BATCH = 16384
_BW = _BATCH // _NW
_NCHF = _BW // _CH
_L = 16

_MESH = plsc.VectorSubcoreMesh(core_axis_name="c", subcore_axis_name="s")


@functools.partial(
    pl.kernel,
    out_type=jax.ShapeDtypeStruct((_V * 32,), jnp.float32),
    mesh=_MESH,
    scratch_types=[
        pltpu.VMEM((32, 128), jnp.float32),
        pltpu.VMEM((32, 128), jnp.float32),
        pltpu.VMEM((4096,), jnp.float32),
        pltpu.VMEM((4096,), jnp.float32),
        pltpu.SemaphoreType.DMA,
        pltpu.SemaphoreType.DMA,
        pltpu.SemaphoreType.DMA,
        pltpu.SemaphoreType.DMA,
    ],
    compiler_params=pltpu.CompilerParams(use_tc_tiling_on_sc=True,
                                         needs_layout_passes=False),
)
def _kt(tbl_t, tail_in, out, src_v0, src_v1, dst_v0, dst_v1,
        si0, si1, so0, so1):
  wid = lax.axis_index("s") * _NC + lax.axis_index("c")
  lane = lax.iota(jnp.int32, _L)
  _SRC = (src_v0, src_v1)
  _DST = (dst_v0, dst_v1)

  def fire_in(j, p, sem):
    pltpu.async_copy(tbl_t.at[:, pl.ds(j * 128, 128)], _SRC[p], sem)

  def transpose_block(p, width):
    src, dst = _SRC[p], _DST[p]

    @plsc.parallel_loop(0, width // _L, unroll=4)
    def _(q):
      for k in range(_L):
        perm = (lane + k) & (_L - 1)
        cvec = q * _L + lane
        dflat = q * 512 + lane * 32 + perm
        for h in range(2):
          vals = plsc.load_gather(src, [h * _L + perm, cvec])
          plsc.store_scatter(dst, [dflat + h * _L], vals)

  def fire_out(j, p, sem, width):
    pltpu.async_copy(_DST[p].at[pl.ds(0, width * 32)],
                     out.at[pl.ds(j * 4096, width * 32)], sem)

  def wait(sem, nbytes):
    pltpu.make_async_copy(out.at[pl.ds(0, nbytes // 4)],
                          dst_v0.at[pl.ds(0, nbytes // 4)], sem).wait()

  fire_in(wid, 0, si0)
  fire_in(wid + _NW, 1, si1)

  def body(i, carry):
    t0 = 2 * i
    wait(si0, 16384)
    transpose_block(0, 128)
    fire_out(wid + _NW * t0, 0, so0, 128)
    wait(si1, 16384)
    transpose_block(1, 128)
    fire_out(wid + _NW * (t0 + 1), 1, so1, 128)
    wait(so0, 16384)
    fire_in(wid + _NW * (t0 + 2), 0, si0)
    wait(so1, 16384)
    fire_in(wid + _NW * (t0 + 3), 1, si1)
    return carry

  lax.fori_loop(0, _BPW // 2 - 1, body, 0)

  wait(si0, 16384)
  transpose_block(0, 128)
  fire_out(wid + _NW * (_BPW - 2), 0, so0, 128)
  wait(si1, 16384)
  transpose_block(1, 128)
  fire_out(wid + _NW * (_BPW - 1), 1, so1, 128)
  wait(so0, 16384)
  wait(so1, 16384)

  @pl.when(wid < _NTAIL)
  def _():
    j = _NW * _BPW + wid
    fire_in(j, 0, si0)
    wait(si0, 16384)
    transpose_block(0, 128)
    fire_out(j, 0, so0, 128)
    wait(so0, 16384)

  @pl.when(wid == _NTAIL)
  def _():
    pltpu.sync_copy(tail_in, dst_v0.at[pl.ds(0, _TAIL * 32)])
    pltpu.sync_copy(dst_v0.at[pl.ds(0, _TAIL * 32)],
                    out.at[pl.ds(_NBLK * 4096, _TAIL * 32)])


@functools.partial(jax.jit, static_argnums=(2,))
def _sc_gather(tbl_rm, idx, d):
  @functools.partial(
      pl.kernel,
      out_type=jax.ShapeDtypeStruct((_NFIELD, d // 8, _BATCH // 128, 8, 128),
                                    jnp.float32),
      mesh=_MESH,
      scratch_types=[
          pltpu.VMEM((_BW, _NFIELD), jnp.int32),
          pltpu.VMEM((_NFIELD, _BW), jnp.int32),
          pltpu.VMEM((_BW, d), jnp.float32),
          pltpu.VMEM((_BW, d), jnp.float32),
          pltpu.VMEM((4, 4, 8, 128), jnp.float32),
          pltpu.VMEM((4, 4, 8, 128), jnp.float32),
          pltpu.SemaphoreType.DMA,
          pltpu.SemaphoreType.DMA,
          pltpu.SemaphoreType.DMA,
          pltpu.SemaphoreType.DMA,
      ],
      compiler_params=pltpu.CompilerParams(use_tc_tiling_on_sc=False,
                                           needs_layout_passes=False),
  )
  def k(table_hbm, idx_hbm, out_hbm, idx_raw, idx_v, rows_v0, rows_v1,
        stg0, stg1, sem0, sem1, wsem0, wsem1):
    _ROWS = (rows_v0, rows_v1)
    _STG = (stg0, stg1)
    wid = lax.axis_index("s") * _NC + lax.axis_index("c")
    base = wid * _BW
    pltpu.sync_copy(idx_hbm.at[pl.ds(base, _BW)], idx_raw)

    lane = lax.iota(jnp.int32, _L)
    for j0 in range(_BW // _L):
      rows = j0 * _L + lane

      def tbody(f, carry):
        cols = jnp.full((_L,), 0, jnp.int32) + f
        vals = plsc.load_gather(idx_raw, [rows, cols])
        idx_v[f, pl.ds(j0 * _L, _L)] = vals
        return carry

      lax.fori_loop(0, _NFIELD, tbody, 0)

    def fire_field(f, p, sem):
      for c in range(_NCHF):
        pltpu.async_copy(
            table_hbm.at[idx_v.at[f, pl.ds(c * _CH, _CH)]],
            _ROWS[p].at[pl.ds(c * _CH, _CH)],
            sem,
        )

    def drain_field(p, sem):
      pltpu.make_async_copy(
          out_hbm.at[0, :, pl.ds(0, 4)], _STG[p], sem).wait()

    def transpose_field(p):
      rows, stg = _ROWS[p], _STG[p]

      @plsc.parallel_loop(0, _BW // _L, unroll=2)
      def _(q):
        bvec = q * _L + lane
        tcl = q // 8
        cvec = (q % 8) * _L + lane
        for k in range(_L):
          perm = (lane + k) & (_L - 1)
          svec = perm % 8
          trb = perm // 8
          for h in range(2):
            vals = plsc.load_gather(rows, [bvec, h * _L + perm])
            plsc.store_scatter(stg, [trb + 2 * h, tcl + svec * 0, svec, cvec],
                               vals)

    def fire_out(f, p, sem):
      for tr in range(4):
        pltpu.async_copy(_STG[p].at[tr],
                         out_hbm.at[f, tr, pl.ds(4 * wid, 4)], sem)

    def wait_out(p, sem):
      pltpu.make_async_copy(out_hbm.at[0, :, pl.ds(0, 4)], _STG[p],
                            sem).wait()

    fire_field(0, 0, sem0)
    fire_field(1, 1, sem1)

    drain_field(0, sem0)
    transpose_field(0)
    fire_out(0, 0, wsem0)
    fire_field(2, 0, sem0)
    drain_field(1, sem1)
    transpose_field(1)
    fire_out(1, 1, wsem1)
    fire_field(3, 1, sem1)

    def body(i, carry):
      f = 2 * i
      drain_field(0, sem0)
      wait_out(0, wsem0)
      transpose_field(0)
      fire_out(f, 0, wsem0)
      fire_field(f + 2, 0, sem0)
      drain_field(1, sem1)
      wait_out(1, wsem1)
      transpose_field(1)
      fire_out(f + 1, 1, wsem1)
      fire_field(f + 3, 1, sem1)
      return carry

    lax.fori_loop(1, _NFIELD // 2 - 1, body, 0)

    drain_field(0, sem0)
    wait_out(0, wsem0)
    transpose_field(0)
    fire_out(_NFIELD - 2, 0, wsem0)
    drain_field(1, sem1)
    wait_out(1, wsem1)
    transpose_field(1)
    fire_out(_NFIELD - 1, 1, wsem1)
    wait_out(0, wsem0)
    wait_out(1, wsem1)

  return k(tbl_rm, idx)


def kernel(inputs, embedding):
  b, f = inputs.shape
  d = embedding.shape[1]
  idx = inputs if inputs.dtype == jnp.int32 else inputs.astype(jnp.int32)
  tail = embedding[_NBLK * 128:].reshape(-1)
  tbl_rm = _kt(embedding.T, tail).reshape(_V, d)
  out5 = _sc_gather(tbl_rm, idx, d)
  out = out5.transpose(0, 1, 3, 2, 4).reshape(f, d, b)
  return out.transpose(2, 0, 1)


# speedup vs baseline: 4.2795x; 1.1301x over previous
import functools

import jax
import jax.numpy as jnp
from jax import lax
from jax.experimental import pallas as pl
from jax.experimental.pallas import tpu as pltpu
from jax.experimental.pallas import tpu_sc as plsc

_NC = 2
_NS = 16
_NW = _NC * _NS

_V = 1000000
_NBLK = _V // 128
_TAIL = _V - _NBLK * 128
_BPW = _NBLK // _NW
_NTAIL = _NBLK - _NW * _BPW

_CH = 128
_NFIELD = 26
_BATCH = 16384
_BW = _BATCH // _NW
_NCHF = _BW // _CH
_L = 16

_MESH = plsc.VectorSubcoreMesh(core_axis_name="c", subcore_axis_name="s")


@functools.partial(
    pl.kernel,
    out_type=jax.ShapeDtypeStruct((_V * 32,), jnp.float32),
    mesh=_MESH,
    scratch_types=[
        pltpu.VMEM((32, 128), jnp.float32),
        pltpu.VMEM((32, 128), jnp.float32),
        pltpu.VMEM((4096,), jnp.float32),
        pltpu.VMEM((4096,), jnp.float32),
        pltpu.SemaphoreType.DMA,
        pltpu.SemaphoreType.DMA,
        pltpu.SemaphoreType.DMA,
        pltpu.SemaphoreType.DMA,
    ],
    compiler_params=pltpu.CompilerParams(use_tc_tiling_on_sc=True,
                                         needs_layout_passes=False),
)
def _kt(tbl_t, tail_in, out, src_v0, src_v1, dst_v0, dst_v1,
        si0, si1, so0, so1):
  wid = lax.axis_index("s") * _NC + lax.axis_index("c")
  lane = lax.iota(jnp.int32, _L)
  _SRC = (src_v0, src_v1)
  _DST = (dst_v0, dst_v1)

  def fire_in(j, p, sem):
    pltpu.async_copy(tbl_t.at[:, pl.ds(j * 128, 128)], _SRC[p], sem)

  def transpose_block(p, width, unr=2):
    src, dst = _SRC[p], _DST[p]

    @plsc.parallel_loop(0, width // _L, unroll=unr)
    def _(q):
      for k in range(_L):
        perm = (lane + k) & (_L - 1)
        cvec = q * _L + lane
        dflat = q * 512 + lane * 32 + perm
        for h in range(2):
          vals = plsc.load_gather(src, [h * _L + perm, cvec])
          plsc.store_scatter(dst, [dflat + h * _L], vals)

  def fire_out(j, p, sem, width):
    pltpu.async_copy(_DST[p].at[pl.ds(0, width * 32)],
                     out.at[pl.ds(j * 4096, width * 32)], sem)

  def wait(sem, nbytes):
    pltpu.make_async_copy(out.at[pl.ds(0, nbytes // 4)],
                          dst_v0.at[pl.ds(0, nbytes // 4)], sem).wait()

  fire_in(wid, 0, si0)
  fire_in(wid + _NW, 1, si1)

  def body(i, carry):
    t0 = 2 * i
    wait(si0, 16384)

    @pl.when(i > 0)
    def _():
      wait(so0, 16384)

    transpose_block(0, 128, unr=4)
    fire_out(wid + _NW * t0, 0, so0, 128)

    @pl.when(t0 + 2 < _BPW)
    def _():
      fire_in(wid + _NW * (t0 + 2), 0, si0)

    wait(si1, 16384)

    @pl.when(i > 0)
    def _():
      wait(so1, 16384)

    transpose_block(1, 128, unr=4)
    fire_out(wid + _NW * (t0 + 1), 1, so1, 128)

    @pl.when(t0 + 3 < _BPW)
    def _():
      fire_in(wid + _NW * (t0 + 3), 1, si1)

    return carry

  lax.fori_loop(0, _BPW // 2, body, 0)
  wait(so0, 16384)
  wait(so1, 16384)

  @pl.when(wid < _NTAIL)
  def _():
    j = _NW * _BPW + wid
    fire_in(j, 0, si0)
    wait(si0, 16384)
    transpose_block(0, 128)
    fire_out(j, 0, so0, 128)
    wait(so0, 16384)

  @pl.when(wid == _NTAIL)
  def _():
    pltpu.sync_copy(tail_in, dst_v0.at[pl.ds(0, _TAIL * 32)])
    pltpu.sync_copy(dst_v0.at[pl.ds(0, _TAIL * 32)],
                    out.at[pl.ds(_NBLK * 4096, _TAIL * 32)])


@functools.partial(jax.jit, static_argnums=(2,))
def _sc_gather(tbl_rm, idx, d):
  @functools.partial(
      pl.kernel,
      out_type=jax.ShapeDtypeStruct((_NFIELD, d // 8, _BATCH // 128, 8, 128),
                                    jnp.float32),
      mesh=_MESH,
      scratch_types=[
          pltpu.VMEM((_BW, _NFIELD), jnp.int32),
          pltpu.VMEM((_NFIELD, _BW), jnp.int32),
          pltpu.VMEM((_BW, d), jnp.float32),
          pltpu.VMEM((_BW, d), jnp.float32),
          pltpu.VMEM((4, 4, 8, 128), jnp.float32),
          pltpu.VMEM((4, 4, 8, 128), jnp.float32),
          pltpu.SemaphoreType.DMA,
          pltpu.SemaphoreType.DMA,
          pltpu.SemaphoreType.DMA,
          pltpu.SemaphoreType.DMA,
      ],
      compiler_params=pltpu.CompilerParams(use_tc_tiling_on_sc=False,
                                           needs_layout_passes=False),
  )
  def k(table_hbm, idx_hbm, out_hbm, idx_raw, idx_v, rows_v0, rows_v1,
        stg0, stg1, sem0, sem1, wsem0, wsem1):
    _ROWS = (rows_v0, rows_v1)
    _STG = (stg0, stg1)
    wid = lax.axis_index("s") * _NC + lax.axis_index("c")
    base = wid * _BW
    pltpu.sync_copy(idx_hbm.at[pl.ds(base, _BW)], idx_raw)

    lane = lax.iota(jnp.int32, _L)
    for j0 in range(_BW // _L):
      rows = j0 * _L + lane

      def tbody(f, carry):
        cols = jnp.full((_L,), 0, jnp.int32) + f
        vals = plsc.load_gather(idx_raw, [rows, cols])
        idx_v[f, pl.ds(j0 * _L, _L)] = vals
        return carry

      lax.fori_loop(0, _NFIELD, tbody, 0)

    def fire_field(f, p, sem):
      for c in range(_NCHF):
        pltpu.async_copy(
            table_hbm.at[idx_v.at[f, pl.ds(c * _CH, _CH)]],
            _ROWS[p].at[pl.ds(c * _CH, _CH)],
            sem,
        )

    def drain_field(p, sem):
      pltpu.make_async_copy(
          out_hbm.at[0, :, pl.ds(0, 4)], _STG[p], sem).wait()

    def transpose_field(p):
      rows, stg = _ROWS[p], _STG[p]

      @plsc.parallel_loop(0, _BW // _L, unroll=2)
      def _(q):
        bvec = q * _L + lane
        tcl = q // 8
        cvec = (q % 8) * _L + lane
        for k in range(_L):
          perm = (lane + k) & (_L - 1)
          svec = perm % 8
          trb = perm // 8
          for h in range(2):
            vals = plsc.load_gather(rows, [bvec, h * _L + perm])
            plsc.store_scatter(stg, [trb + 2 * h, tcl + svec * 0, svec, cvec],
                               vals)

    def fire_out(f, p, sem):
      for tr in range(4):
        pltpu.async_copy(_STG[p].at[tr],
                         out_hbm.at[f, tr, pl.ds(4 * wid, 4)], sem)

    def wait_out(p, sem):
      pltpu.make_async_copy(out_hbm.at[0, :, pl.ds(0, 4)], _STG[p],
                            sem).wait()

    fire_field(0, 0, sem0)
    fire_field(1, 1, sem1)

    drain_field(0, sem0)
    transpose_field(0)
    fire_out(0, 0, wsem0)
    fire_field(2, 0, sem0)
    drain_field(1, sem1)
    transpose_field(1)
    fire_out(1, 1, wsem1)
    fire_field(3, 1, sem1)

    def body(i, carry):
      f = 2 * i
      drain_field(0, sem0)
      wait_out(0, wsem0)
      transpose_field(0)
      fire_out(f, 0, wsem0)
      fire_field(f + 2, 0, sem0)
      drain_field(1, sem1)
      wait_out(1, wsem1)
      transpose_field(1)
      fire_out(f + 1, 1, wsem1)
      fire_field(f + 3, 1, sem1)
      return carry

    lax.fori_loop(1, _NFIELD // 2 - 1, body, 0)

    drain_field(0, sem0)
    wait_out(0, wsem0)
    transpose_field(0)
    fire_out(_NFIELD - 2, 0, wsem0)
    drain_field(1, sem1)
    wait_out(1, wsem1)
    transpose_field(1)
    fire_out(_NFIELD - 1, 1, wsem1)
    wait_out(0, wsem0)
    wait_out(1, wsem1)

  return k(tbl_rm, idx)


def kernel(inputs, embedding):
  b, f = inputs.shape
  d = embedding.shape[1]
  idx = inputs if inputs.dtype == jnp.int32 else inputs.astype(jnp.int32)
  tail = embedding[_NBLK * 128:].reshape(-1)
  tbl_rm = _kt(embedding.T, tail).reshape(_V, d)
  out5 = _sc_gather(tbl_rm, idx, d)
  out = out5.transpose(0, 1, 3, 2, 4).reshape(f, d, b)
  return out.transpose(2, 0, 1)
